# Initial kernel scaffold; baseline (speedup 1.0000x reference)
import functools
import math

import jax
import jax.numpy as jnp
from jax import lax
from jax.experimental import pallas as pl
from jax.experimental.pallas import tpu as pltpu
from jax.experimental.pallas import tpu_sc as plsc

N = 10000
E = 320000
D = 128
ALPHA_C = 0.1
BETA_C = float(math.log(1.5))

NC = 2
NS = 16
NW = NC * NS
EPW = E // NW
CH = 80
NCH = EPW // CH
RPT = N // NS
RCH = 125
LANES = 16


def _sc_body(src_ref, dst_ref, w_ref, x_ref, out_ref,
             src_v, dst_v, w_v, rows_v, cbuf, agg_sh, sem):
    c = lax.axis_index("c")
    s = lax.axis_index("s")
    wid = c * NS + s

    z16 = jnp.zeros((LANES,), jnp.float32)

    def zero_row(i, _):
        for k in range(D // LANES):
            cbuf[i, pl.ds(k * LANES, LANES)] = z16
        return 0

    lax.fori_loop(0, RCH, zero_row, 0)
    r0 = s * RPT
    for t in range(RPT // RCH):
        pltpu.sync_copy(cbuf, agg_sh.at[pl.ds(r0 + t * RCH, RCH)])
    plsc.subcore_barrier()

    pltpu.sync_copy(src_ref.at[pl.ds(wid * NCH, NCH)], src_v)
    pltpu.sync_copy(dst_ref.at[pl.ds(wid * NCH, NCH)], dst_v)
    pltpu.sync_copy(w_ref.at[pl.ds(wid * NCH, NCH)], w_v)

    def chunk(j, _):
        pltpu.async_copy(x_ref.at[src_v.at[j]], rows_v, sem).wait()

        def scale(i, _):
            wv = w_v[j, i]
            for k in range(D // LANES):
                sl = pl.ds(k * LANES, LANES)
                rows_v[i, sl] = rows_v[i, sl] * wv
            return 0

        lax.fori_loop(0, CH, scale, 0)

        pltpu.sync_copy(rows_v, agg_sh.at[dst_v.at[j]], add=True)
        return 0

    lax.fori_loop(0, NCH, chunk, 0)
    plsc.subcore_barrier()

    for t in range(RPT // RCH):
        rr = r0 + t * RCH
        pltpu.sync_copy(agg_sh.at[pl.ds(rr, RCH)], cbuf)
        pltpu.sync_copy(cbuf, out_ref.at[c, pl.ds(rr, RCH)])


@functools.partial(
    pl.kernel,
    out_type=jax.ShapeDtypeStruct((NC, N, D), jnp.float32),
    mesh=plsc.VectorSubcoreMesh(core_axis_name="c", subcore_axis_name="s"),
    scratch_types=[
        pltpu.VMEM((NCH, CH), jnp.int32),
        pltpu.VMEM((NCH, CH), jnp.int32),
        pltpu.VMEM((NCH, CH), jnp.float32),
        pltpu.VMEM((CH, D), jnp.float32),
        pltpu.VMEM((RCH, D), jnp.float32),
        pltpu.VMEM_SHARED((N, D), jnp.float32),
        pltpu.SemaphoreType.DMA,
    ],
)
def _sc_scatter(src_ref, dst_ref, w_ref, x_ref, out_ref,
                src_v, dst_v, w_v, rows_v, cbuf, agg_sh, sem):
    _sc_body(src_ref, dst_ref, w_ref, x_ref, out_ref,
             src_v, dst_v, w_v, rows_v, cbuf, agg_sh, sem)


def _tc_body(p_ref, x0_ref, w_ref, o_ref):
    t = (1.0 - ALPHA_C) * (p_ref[0] + p_ref[1]) + ALPHA_C * x0_ref[...]
    o_ref[...] = (1.0 - BETA_C) * t + BETA_C * jnp.dot(
        t, w_ref[...], preferred_element_type=jnp.float32)


BN = 2000


def _tc_combine(partials, x_0, weight1):
    return pl.pallas_call(
        _tc_body,
        grid=(N // BN,),
        in_specs=[
            pl.BlockSpec((NC, BN, D), lambda i: (0, i, 0)),
            pl.BlockSpec((BN, D), lambda i: (i, 0)),
            pl.BlockSpec((D, D), lambda i: (0, 0)),
        ],
        out_specs=pl.BlockSpec((BN, D), lambda i: (i, 0)),
        out_shape=jax.ShapeDtypeStruct((N, D), jnp.float32),
    )(partials, x_0, weight1)


def kernel(x, edge_index, edge_weight, x_0, weight1):
    src = edge_index[0].reshape(NW * NCH, CH)
    dst = edge_index[1].reshape(NW * NCH, CH)
    w2 = edge_weight.reshape(NW * NCH, CH)
    partials = _sc_scatter(src, dst, w2, x)
    return _tc_combine(partials, x_0, weight1)


# speedup vs baseline: 3.0867x; 3.0867x over previous
import functools
import math

import jax
import jax.numpy as jnp
from jax import lax
from jax.experimental import pallas as pl
from jax.experimental.pallas import tpu as pltpu
from jax.experimental.pallas import tpu_sc as plsc

N = 10000
E = 320000
D = 128
HD = D // 2
ALPHA_C = 0.1
BETA_C = float(math.log(1.5))

NC = 2
NS = 16
EPT = E // NS
CH = 80
NCH = EPT // CH
RPT = 624
RCH = 208
TAIL = N - NS * RPT
LANES = 16


def _sc_body(src_ref, dst_ref, w_ref, xa_ref, xb_ref, out_ref,
             src_v, dst_v, w_v, dst80_v, rows_v, cbuf, agg_sh, sem):
    c = lax.axis_index("c")
    s = lax.axis_index("s")

    z16 = jnp.zeros((LANES,), jnp.float32)

    def zero_row(i, _):
        for k in range(HD // LANES):
            cbuf[i, pl.ds(k * LANES, LANES)] = z16
        return 0

    lax.fori_loop(0, RCH, zero_row, 0)
    r0 = s * RPT
    for t in range(RPT // RCH):
        pltpu.sync_copy(cbuf, agg_sh.at[pl.ds(r0 + t * RCH, RCH)])

    @pl.when(s == NS - 1)
    def _zero_tail():
        pltpu.sync_copy(cbuf.at[pl.ds(0, TAIL)],
                        agg_sh.at[pl.ds(NS * RPT, TAIL)])

    plsc.subcore_barrier()

    pltpu.sync_copy(src_ref.at[pl.ds(s * EPT, EPT)], src_v)
    pltpu.sync_copy(dst_ref.at[pl.ds(s * EPT, EPT)], dst_v)
    pltpu.sync_copy(w_ref.at[pl.ds(s * EPT, EPT)], w_v)

    def make_chunk(x_half_ref):
        def chunk(j, _):
            e0 = j * CH
            pltpu.async_copy(x_half_ref.at[src_v.at[pl.ds(e0, CH)]],
                             rows_v, sem).wait()
            for g in range(CH // LANES):
                dst80_v[pl.ds(g * LANES, LANES)] = (
                    dst_v[pl.ds(e0 + g * LANES, LANES)])

            def scale(g, _):
                w16 = w_v[pl.ds(e0 + g * LANES, LANES)]
                for l in range(LANES):
                    i = g * LANES + l
                    wv = w16[l]
                    for k in range(HD // LANES):
                        sl = pl.ds(k * LANES, LANES)
                        rows_v[i, sl] = rows_v[i, sl] * wv
                return 0

            lax.fori_loop(0, CH // LANES, scale, 0)

            pltpu.sync_copy(rows_v, agg_sh.at[dst80_v], add=True)
            return 0

        return chunk

    @pl.when(c == 0)
    def _run_a():
        lax.fori_loop(0, NCH, make_chunk(xa_ref), 0)

    @pl.when(c == 1)
    def _run_b():
        lax.fori_loop(0, NCH, make_chunk(xb_ref), 0)

    plsc.subcore_barrier()

    for t in range(RPT // RCH):
        rr = r0 + t * RCH
        pltpu.sync_copy(agg_sh.at[pl.ds(rr, RCH)], cbuf)
        pltpu.sync_copy(cbuf, out_ref.at[c, pl.ds(rr, RCH)])

    @pl.when(s == NS - 1)
    def _write_tail():
        pltpu.sync_copy(agg_sh.at[pl.ds(NS * RPT, TAIL)],
                        cbuf.at[pl.ds(0, TAIL)])
        pltpu.sync_copy(cbuf.at[pl.ds(0, TAIL)],
                        out_ref.at[c, pl.ds(NS * RPT, TAIL)])


@functools.partial(
    pl.kernel,
    out_type=jax.ShapeDtypeStruct((NC, N, HD), jnp.float32),
    mesh=plsc.VectorSubcoreMesh(core_axis_name="c", subcore_axis_name="s"),
    compiler_params=pltpu.CompilerParams(use_tc_tiling_on_sc=False),
    scratch_types=[
        pltpu.VMEM((EPT,), jnp.int32),
        pltpu.VMEM((EPT,), jnp.int32),
        pltpu.VMEM((EPT,), jnp.float32),
        pltpu.VMEM((CH,), jnp.int32),
        pltpu.VMEM((CH, HD), jnp.float32),
        pltpu.VMEM((RCH, HD), jnp.float32),
        pltpu.VMEM_SHARED((N, HD), jnp.float32),
        pltpu.SemaphoreType.DMA,
    ],
)
def _sc_scatter(src_ref, dst_ref, w_ref, xa_ref, xb_ref, out_ref,
                src_v, dst_v, w_v, dst80_v, rows_v, cbuf, agg_sh, sem):
    _sc_body(src_ref, dst_ref, w_ref, xa_ref, xb_ref, out_ref,
             src_v, dst_v, w_v, dst80_v, rows_v, cbuf, agg_sh, sem)


def _tc_body(p_ref, x0_ref, w_ref, o_ref):
    agg = jnp.concatenate([p_ref[0], p_ref[1]], axis=-1)
    t = (1.0 - ALPHA_C) * agg + ALPHA_C * x0_ref[...]
    o_ref[...] = (1.0 - BETA_C) * t + BETA_C * jnp.dot(
        t, w_ref[...], preferred_element_type=jnp.float32)


BN = 2000


def _tc_combine(partials, x_0, weight1):
    return pl.pallas_call(
        _tc_body,
        grid=(N // BN,),
        in_specs=[
            pl.BlockSpec((NC, BN, HD), lambda i: (0, i, 0)),
            pl.BlockSpec((BN, D), lambda i: (i, 0)),
            pl.BlockSpec((D, D), lambda i: (0, 0)),
        ],
        out_specs=pl.BlockSpec((BN, D), lambda i: (i, 0)),
        out_shape=jax.ShapeDtypeStruct((N, D), jnp.float32),
    )(partials, x_0, weight1)


def kernel(x, edge_index, edge_weight, x_0, weight1):
    src = edge_index[0]
    dst = edge_index[1]
    xa = x[:, :HD]
    xb = x[:, HD:]
    partials = _sc_scatter(src, dst, edge_weight, xa, xb)
    return _tc_combine(partials, x_0, weight1)
